# Initial kernel scaffold; baseline (speedup 1.0000x reference)
import jax
import jax.numpy as jnp
from jax.experimental import pallas as pl


def kernel(x):
    raise NotImplementedError("write your pallas kernel here")


# speedup vs baseline: 10.4688x; 10.4688x over previous
import jax
import jax.numpy as jnp
from jax import lax
from jax.experimental import pallas as pl
from jax.experimental.pallas import tpu as pltpu
from jax.experimental.pallas import tpu_sc as plsc

N = 128 * 32768
NTOP = 64 * 128
NC, NS, L = 2, 16, 16
NW = NC * NS
PER_W = N // NW
CHUNK = 8192
VECS = CHUNK // L
NCHUNK = PER_W // CHUNK

NB1, NB2, NB3 = 2048, 1024, 1024
SH1, SH2 = 20, 10

_MESH = plsc.VectorSubcoreMesh(
    core_axis_name="c", subcore_axis_name="s", num_cores=NC, num_subcores=NS)


def _iota():
    return lax.iota(jnp.int32, L)


def _at_lane(vec, lane):
    return jnp.sum(jnp.where(_iota() == lane, vec, jnp.zeros((L,), vec.dtype)))


def _worker_id():
    return lax.axis_index("s") * NC + lax.axis_index("c")


def _zero_hist(ref, nb):
    z = jnp.zeros((L,), jnp.int32)

    def body(j, _):
        ref[pl.ds(j * L, L)] = z
        return 0

    lax.fori_loop(0, nb // L, body, 0)


def _merge_rows(all_ref, hm_ref, nb):
    def jbody(jb, _):
        def wbody(w, acc):
            return acc + all_ref[w, pl.ds(jb * L, L)]

        acc = lax.fori_loop(0, NW, wbody, jnp.zeros((L,), jnp.int32))
        hm_ref[pl.ds(jb * L, L)] = acc
        return 0

    lax.fori_loop(0, nb // L, jbody, 0)


def _find_cross(hm_ref, nb, r):
    iot = _iota()

    def body(i, car):
        b, g, cum = car
        j = nb // L - 1 - i
        v = hm_ref[pl.ds(j * L, L)]
        rv = lax.rev(v, dimensions=(0,))
        cs = plsc.cumsum(rv)
        total = jnp.sum(v)
        cross = (cs + cum) >= r
        anyc = jnp.any(cross)
        lane = jnp.min(jnp.where(cross, iot, jnp.full((L,), L, jnp.int32)))
        csl = _at_lane(cs, lane)
        rvl = _at_lane(rv, lane)
        newly = jnp.logical_and(anyc, b < 0)
        b = jnp.where(newly, j * L + (L - 1) - lane, b)
        g = jnp.where(newly, cum + csl - rvl, g)
        return (b, g, cum + total)

    b, g, _ = lax.fori_loop(
        0, nb // L, body,
        (jnp.int32(-1), jnp.int32(0), jnp.int32(0)))
    return b, g


def _hist_pass(x_ref, dbuf, hist_ref, base, bucket_fn):
    ones = jnp.ones((L,), jnp.int32)

    def cbody(i, _):
        pltpu.sync_copy(x_ref.at[pl.ds(base + i * CHUNK, CHUNK)], dbuf)

        def vbody(t, _2):
            v = dbuf[pl.ds(t * L, L)]
            k = lax.bitcast_convert_type(v, jnp.int32)
            idx, m = bucket_fn(k)
            plsc.addupdate_scatter(hist_ref, [idx], ones, mask=m)
            return 0

        lax.fori_loop(0, VECS, vbody, 0)
        return 0

    lax.fori_loop(0, NCHUNK, cbody, 0)


def _scalars_to_vec(vals):
    iot = _iota()
    out = jnp.zeros((L,), jnp.int32)
    for i, v in enumerate(vals):
        out = jnp.where(iot == i, v, out)
    return out


def _l1_body(x_ref, h1_ref, dbuf, hist):
    wid = _worker_id()
    _zero_hist(hist, NB1)

    def bucket(k):
        uk = jnp.maximum(k, 0)
        return uk >> SH1, k > 0

    _hist_pass(x_ref, dbuf, hist, wid * PER_W, bucket)
    pltpu.sync_copy(hist, h1_ref.at[wid])


def _l2_body(x_ref, h1_ref, h2_ref, s1_ref, dbuf, allbuf, hm, hist, svec):
    wid = _worker_id()
    pltpu.sync_copy(h1_ref, allbuf)
    _merge_rows(allbuf, hm, NB1)
    b1, g1 = _find_cross(hm, NB1, jnp.int32(NTOP))

    _zero_hist(hist, NB2)

    def bucket(k):
        uk = jnp.maximum(k, 0)
        m = jnp.logical_and(uk >> SH1 == b1, k > 0)
        return (uk >> SH2) & (NB2 - 1), m

    _hist_pass(x_ref, dbuf, hist, wid * PER_W, bucket)
    pltpu.sync_copy(hist, h2_ref.at[wid])

    svec[...] = _scalars_to_vec([b1, g1])

    @pl.when(wid == 0)
    def _():
        pltpu.sync_copy(svec, s1_ref)


def _l3_body(x_ref, h2_ref, s1_ref, h3_ref, s2_ref, dbuf, allbuf, hm, hist,
             svec):
    wid = _worker_id()
    pltpu.sync_copy(s1_ref, svec)
    sv = svec[...]
    b1, g1 = _at_lane(sv, 0), _at_lane(sv, 1)

    pltpu.sync_copy(h2_ref, allbuf)
    _merge_rows(allbuf, hm, NB2)
    b2, g2i = _find_cross(hm, NB2, NTOP - g1)
    g2 = g1 + g2i

    _zero_hist(hist, NB3)
    pref = (b1 << (SH1 - SH2)) | b2

    def bucket(k):
        uk = jnp.maximum(k, 0)
        m = jnp.logical_and(uk >> SH2 == pref, k > 0)
        return uk & (NB3 - 1), m

    _hist_pass(x_ref, dbuf, hist, wid * PER_W, bucket)
    pltpu.sync_copy(hist, h3_ref.at[wid])

    svec[...] = _scalars_to_vec([b2, g2])

    @pl.when(wid == 0)
    def _():
        pltpu.sync_copy(svec, s2_ref)


def _l4_body(x_ref, s1_ref, s2_ref, h3_ref, out_ref, dbuf, obuf, allbuf, hm,
             svec):
    wid = _worker_id()
    base = wid * PER_W

    pltpu.sync_copy(s1_ref, svec)
    sv = svec[...]
    b1 = _at_lane(sv, 0)
    pltpu.sync_copy(s2_ref, svec)
    sv = svec[...]
    b2, g2 = _at_lane(sv, 0), _at_lane(sv, 1)

    pltpu.sync_copy(h3_ref, allbuf)
    _merge_rows(allbuf, hm, NB3)
    b3, g3i = _find_cross(hm, NB3, NTOP - g2)
    g = g2 + g3i

    t_key = (b1 << SH1) | (b2 << SH2) | b3
    vb3 = hm[pl.ds((b3 >> 4) << 4, L)]
    e_cnt = _at_lane(vb3, b3 & (L - 1))
    quota = NTOP - g

    def bbody(w, acc):
        vw = allbuf[w, pl.ds((b3 >> 4) << 4, L)]
        ew = _at_lane(vw, b3 & (L - 1))
        return acc + jnp.where(w < wid, ew, 0)

    tie_base = lax.fori_loop(0, NW, bbody, jnp.int32(0))

    simple = jnp.logical_or(b1 < 0, e_cnt == quota)
    t_eff = jnp.where(b1 < 0, jnp.int32(1), t_key)

    @pl.when(simple)
    def _():
        def cbody(i, _):
            off = base + i * CHUNK
            pltpu.sync_copy(x_ref.at[pl.ds(off, CHUNK)], dbuf)

            def vbody(t, _2):
                v = dbuf[pl.ds(t * L, L)]
                k = lax.bitcast_convert_type(v, jnp.int32)
                obuf[pl.ds(t * L, L)] = jnp.where(
                    k >= t_eff, v, jnp.zeros((L,), jnp.float32))
                return 0

            lax.fori_loop(0, VECS, vbody, 0)
            pltpu.sync_copy(obuf, out_ref.at[pl.ds(off, CHUNK)])
            return 0

        lax.fori_loop(0, NCHUNK, cbody, 0)

    @pl.when(jnp.logical_not(simple))
    def _():
        def cbody(i, cnt):
            off = base + i * CHUNK
            pltpu.sync_copy(x_ref.at[pl.ds(off, CHUNK)], dbuf)

            def vbody(t, c):
                v = dbuf[pl.ds(t * L, L)]
                k = lax.bitcast_convert_type(v, jnp.int32)
                gt = k > t_key
                eq = k == t_key
                cs = plsc.cumsum(eq.astype(jnp.int32))
                keep_t = jnp.logical_and(eq, (tie_base + c + cs) <= quota)
                obuf[pl.ds(t * L, L)] = jnp.where(
                    jnp.logical_or(gt, keep_t), v, jnp.zeros((L,), jnp.float32))
                return c + _at_lane(cs, L - 1)

            cnt = lax.fori_loop(0, VECS, vbody, cnt)
            pltpu.sync_copy(obuf, out_ref.at[pl.ds(off, CHUNK)])
            return cnt

        lax.fori_loop(0, NCHUNK, cbody, jnp.int32(0))


def _i32(*shape):
    return jax.ShapeDtypeStruct(shape, jnp.int32)


_PARAMS = pltpu.CompilerParams(needs_layout_passes=False)

_l1 = pl.kernel(
    _l1_body, out_type=_i32(NW, NB1), mesh=_MESH,
    compiler_params=_PARAMS,
    scratch_types=[pltpu.VMEM((CHUNK,), jnp.float32),
                   pltpu.VMEM((NB1,), jnp.int32)])

_l2 = pl.kernel(
    _l2_body, compiler_params=_PARAMS, out_type=(_i32(NW, NB2), _i32(L)), mesh=_MESH,
    scratch_types=[pltpu.VMEM((CHUNK,), jnp.float32),
                   pltpu.VMEM((NW, NB1), jnp.int32),
                   pltpu.VMEM((NB1,), jnp.int32),
                   pltpu.VMEM((NB2,), jnp.int32),
                   pltpu.VMEM((L,), jnp.int32)])

_l3 = pl.kernel(
    _l3_body, compiler_params=_PARAMS, out_type=(_i32(NW, NB3), _i32(L)), mesh=_MESH,
    scratch_types=[pltpu.VMEM((CHUNK,), jnp.float32),
                   pltpu.VMEM((NW, NB2), jnp.int32),
                   pltpu.VMEM((NB2,), jnp.int32),
                   pltpu.VMEM((NB3,), jnp.int32),
                   pltpu.VMEM((L,), jnp.int32)])

_l4 = pl.kernel(
    _l4_body, compiler_params=_PARAMS,
    out_type=jax.ShapeDtypeStruct((N,), jnp.float32), mesh=_MESH,
    scratch_types=[pltpu.VMEM((CHUNK,), jnp.float32),
                   pltpu.VMEM((CHUNK,), jnp.float32),
                   pltpu.VMEM((NW, NB3), jnp.int32),
                   pltpu.VMEM((NB3,), jnp.int32),
                   pltpu.VMEM((L,), jnp.int32)])


def kernel(x):
    xf = x.reshape(N)
    h1 = _l1(xf)
    h2, s1 = _l2(xf, h1)
    h3, s2 = _l3(xf, h2, s1)
    out = _l4(xf, s1, s2, h3)
    return out.reshape(x.shape)


# speedup vs baseline: 21.1743x; 2.0226x over previous
import jax
import jax.numpy as jnp
from jax import lax
from jax.experimental import pallas as pl
from jax.experimental.pallas import tpu as pltpu
from jax.experimental.pallas import tpu_sc as plsc

N = 128 * 32768
NTOP = 64 * 128
NC, NS, L = 2, 16, 16
NW = NC * NS
PER_W = N // NW
CHUNK = 8192
VECS = CHUNK // L
NCHUNK = PER_W // CHUNK

NB1, NB2, NB3 = 2048, 1024, 1024
SH1, SH2 = 20, 10

_MESH = plsc.VectorSubcoreMesh(
    core_axis_name="c", subcore_axis_name="s", num_cores=NC, num_subcores=NS)


def _iota():
    return lax.iota(jnp.int32, L)


def _at_lane(vec, lane):
    return jnp.sum(jnp.where(_iota() == lane, vec, jnp.zeros((L,), vec.dtype)))


def _worker_id():
    return lax.axis_index("s") * NC + lax.axis_index("c")


def _zero_hist(ref, nb):
    z = jnp.zeros((L,), jnp.int32)

    @plsc.parallel_loop(0, nb // L, unroll=8)
    def _(j):
        ref[pl.ds(j * L, L)] = z


def _merge_rows(all_ref, hm_ref, nb):
    @plsc.parallel_loop(0, nb // L, unroll=2)
    def _(jb):
        acc = jnp.zeros((L,), jnp.int32)
        for w in range(NW):
            acc = acc + all_ref[w, pl.ds(jb * L, L)]
        hm_ref[pl.ds(jb * L, L)] = acc


def _find_cross(hm_ref, nb, r):
    iot = _iota()

    def body(i, car):
        b, g, cum = car
        j = nb // L - 1 - i
        v = hm_ref[pl.ds(j * L, L)]
        rv = lax.rev(v, dimensions=(0,))
        cs = plsc.cumsum(rv)
        total = jnp.sum(v)
        cross = (cs + cum) >= r
        anyc = jnp.any(cross)
        lane = jnp.min(jnp.where(cross, iot, jnp.full((L,), L, jnp.int32)))
        csl = _at_lane(cs, lane)
        rvl = _at_lane(rv, lane)
        newly = jnp.logical_and(anyc, b < 0)
        b = jnp.where(newly, j * L + (L - 1) - lane, b)
        g = jnp.where(newly, cum + csl - rvl, g)
        return (b, g, cum + total)

    b, g, _ = lax.fori_loop(
        0, nb // L, body,
        (jnp.int32(-1), jnp.int32(0), jnp.int32(0)))
    return b, g


def _hist_pass(x_ref, dbuf, hist_ref, base, bucket_fn):
    ones = jnp.ones((L,), jnp.int32)

    def cbody(i, _):
        pltpu.sync_copy(x_ref.at[pl.ds(base + i * CHUNK, CHUNK)], dbuf)

        @plsc.parallel_loop(0, VECS, unroll=8)
        def _(t):
            v = dbuf[pl.ds(t * L, L)]
            k = lax.bitcast_convert_type(v, jnp.int32)
            idx, m = bucket_fn(k)
            plsc.addupdate_scatter(hist_ref, [idx], ones, mask=m)

        return 0

    lax.fori_loop(0, NCHUNK, cbody, 0)


def _scalars_to_vec(vals):
    iot = _iota()
    out = jnp.zeros((L,), jnp.int32)
    for i, v in enumerate(vals):
        out = jnp.where(iot == i, v, out)
    return out


def _l1_body(x_ref, h1_ref, dbuf, hist):
    wid = _worker_id()
    _zero_hist(hist, NB1)

    def bucket(k):
        uk = jnp.maximum(k, 0)
        return uk >> SH1, k > 0

    _hist_pass(x_ref, dbuf, hist, wid * PER_W, bucket)
    pltpu.sync_copy(hist, h1_ref.at[wid])


def _l2_body(x_ref, h1_ref, h2_ref, s1_ref, dbuf, allbuf, hm, hist, svec):
    wid = _worker_id()
    pltpu.sync_copy(h1_ref, allbuf)
    _merge_rows(allbuf, hm, NB1)
    b1, g1 = _find_cross(hm, NB1, jnp.int32(NTOP))

    _zero_hist(hist, NB2)

    def bucket(k):
        uk = jnp.maximum(k, 0)
        m = jnp.logical_and(uk >> SH1 == b1, k > 0)
        return (uk >> SH2) & (NB2 - 1), m

    _hist_pass(x_ref, dbuf, hist, wid * PER_W, bucket)
    pltpu.sync_copy(hist, h2_ref.at[wid])

    svec[...] = _scalars_to_vec([b1, g1])

    @pl.when(wid == 0)
    def _():
        pltpu.sync_copy(svec, s1_ref)


def _l3_body(x_ref, h2_ref, s1_ref, h3_ref, s2_ref, dbuf, allbuf, hm, hist,
             svec):
    wid = _worker_id()
    pltpu.sync_copy(s1_ref, svec)
    sv = svec[...]
    b1, g1 = _at_lane(sv, 0), _at_lane(sv, 1)

    pltpu.sync_copy(h2_ref, allbuf)
    _merge_rows(allbuf, hm, NB2)
    b2, g2i = _find_cross(hm, NB2, NTOP - g1)
    g2 = g1 + g2i

    _zero_hist(hist, NB3)
    pref = (b1 << (SH1 - SH2)) | b2

    def bucket(k):
        uk = jnp.maximum(k, 0)
        m = jnp.logical_and(uk >> SH2 == pref, k > 0)
        return uk & (NB3 - 1), m

    _hist_pass(x_ref, dbuf, hist, wid * PER_W, bucket)
    pltpu.sync_copy(hist, h3_ref.at[wid])

    svec[...] = _scalars_to_vec([b2, g2])

    @pl.when(wid == 0)
    def _():
        pltpu.sync_copy(svec, s2_ref)


def _l4_body(x_ref, s1_ref, s2_ref, h3_ref, out_ref, dbuf, obuf, allbuf, hm,
             svec):
    wid = _worker_id()
    base = wid * PER_W

    pltpu.sync_copy(s1_ref, svec)
    sv = svec[...]
    b1 = _at_lane(sv, 0)
    pltpu.sync_copy(s2_ref, svec)
    sv = svec[...]
    b2, g2 = _at_lane(sv, 0), _at_lane(sv, 1)

    pltpu.sync_copy(h3_ref, allbuf)
    _merge_rows(allbuf, hm, NB3)
    b3, g3i = _find_cross(hm, NB3, NTOP - g2)
    g = g2 + g3i

    t_key = (b1 << SH1) | (b2 << SH2) | b3
    vb3 = hm[pl.ds((b3 >> 4) << 4, L)]
    e_cnt = _at_lane(vb3, b3 & (L - 1))
    quota = NTOP - g

    def bbody(w, acc):
        vw = allbuf[w, pl.ds((b3 >> 4) << 4, L)]
        ew = _at_lane(vw, b3 & (L - 1))
        return acc + jnp.where(w < wid, ew, 0)

    tie_base = lax.fori_loop(0, NW, bbody, jnp.int32(0))

    simple = jnp.logical_or(b1 < 0, e_cnt == quota)
    t_eff = jnp.where(b1 < 0, jnp.int32(1), t_key)

    @pl.when(simple)
    def _():
        def cbody(i, _):
            off = base + i * CHUNK
            pltpu.sync_copy(x_ref.at[pl.ds(off, CHUNK)], dbuf)

            @plsc.parallel_loop(0, VECS, unroll=8)
            def _(t):
                v = dbuf[pl.ds(t * L, L)]
                k = lax.bitcast_convert_type(v, jnp.int32)
                obuf[pl.ds(t * L, L)] = jnp.where(
                    k >= t_eff, v, jnp.zeros((L,), jnp.float32))

            pltpu.sync_copy(obuf, out_ref.at[pl.ds(off, CHUNK)])
            return 0

        lax.fori_loop(0, NCHUNK, cbody, 0)

    @pl.when(jnp.logical_not(simple))
    def _():
        def cbody(i, cnt):
            off = base + i * CHUNK
            pltpu.sync_copy(x_ref.at[pl.ds(off, CHUNK)], dbuf)

            def vbody(t, c):
                v = dbuf[pl.ds(t * L, L)]
                k = lax.bitcast_convert_type(v, jnp.int32)
                gt = k > t_key
                eq = k == t_key
                cs = plsc.cumsum(eq.astype(jnp.int32))
                keep_t = jnp.logical_and(eq, (tie_base + c + cs) <= quota)
                obuf[pl.ds(t * L, L)] = jnp.where(
                    jnp.logical_or(gt, keep_t), v, jnp.zeros((L,), jnp.float32))
                return c + _at_lane(cs, L - 1)

            cnt = lax.fori_loop(0, VECS, vbody, cnt)
            pltpu.sync_copy(obuf, out_ref.at[pl.ds(off, CHUNK)])
            return cnt

        lax.fori_loop(0, NCHUNK, cbody, jnp.int32(0))


def _i32(*shape):
    return jax.ShapeDtypeStruct(shape, jnp.int32)


_PARAMS = pltpu.CompilerParams(needs_layout_passes=False)

_l1 = pl.kernel(
    _l1_body, out_type=_i32(NW, NB1), mesh=_MESH,
    compiler_params=_PARAMS,
    scratch_types=[pltpu.VMEM((CHUNK,), jnp.float32),
                   pltpu.VMEM((NB1,), jnp.int32)])

_l2 = pl.kernel(
    _l2_body, compiler_params=_PARAMS, out_type=(_i32(NW, NB2), _i32(L)), mesh=_MESH,
    scratch_types=[pltpu.VMEM((CHUNK,), jnp.float32),
                   pltpu.VMEM((NW, NB1), jnp.int32),
                   pltpu.VMEM((NB1,), jnp.int32),
                   pltpu.VMEM((NB2,), jnp.int32),
                   pltpu.VMEM((L,), jnp.int32)])

_l3 = pl.kernel(
    _l3_body, compiler_params=_PARAMS, out_type=(_i32(NW, NB3), _i32(L)), mesh=_MESH,
    scratch_types=[pltpu.VMEM((CHUNK,), jnp.float32),
                   pltpu.VMEM((NW, NB2), jnp.int32),
                   pltpu.VMEM((NB2,), jnp.int32),
                   pltpu.VMEM((NB3,), jnp.int32),
                   pltpu.VMEM((L,), jnp.int32)])

_l4 = pl.kernel(
    _l4_body, compiler_params=_PARAMS,
    out_type=jax.ShapeDtypeStruct((N,), jnp.float32), mesh=_MESH,
    scratch_types=[pltpu.VMEM((CHUNK,), jnp.float32),
                   pltpu.VMEM((CHUNK,), jnp.float32),
                   pltpu.VMEM((NW, NB3), jnp.int32),
                   pltpu.VMEM((NB3,), jnp.int32),
                   pltpu.VMEM((L,), jnp.int32)])


def kernel(x):
    xf = x.reshape(N)
    h1 = _l1(xf)
    h2, s1 = _l2(xf, h1)
    h3, s2 = _l3(xf, h2, s1)
    out = _l4(xf, s1, s2, h3)
    return out.reshape(x.shape)
